# Initial kernel scaffold; baseline (speedup 1.0000x reference)
import jax
import jax.numpy as jnp
from jax.experimental import pallas as pl


def kernel(x, edge_index, hyperedge_weight, W1, att1, b1, W2, b2):
    raise NotImplementedError("write your pallas kernel here")


# speedup vs baseline: 73.4198x; 73.4198x over previous
import jax
import jax.numpy as jnp
from jax import lax
from jax.experimental import pallas as pl
from jax.experimental.pallas import tpu as pltpu
from jax.experimental.pallas import tpu_sc as plsc

_N = 10000
_K = 320000
_HEADS = 8
_HID = 8
_NCLS = 7

_NC = 2
_NS = 16
_NW = _NC * _NS
_EPW = _K // _NW
_CB = 80
_NCH = _EPW // _CB
_RPS = 624
_TAIL = _N - _NS * _RPS

_mesh = plsc.VectorSubcoreMesh(
    core_axis_name="c", subcore_axis_name="s", num_cores=_NC, num_subcores=_NS
)

_f32 = jnp.float32


def _wid_cid_sid():
    cid = lax.axis_index("c")
    sid = lax.axis_index("s")
    return sid * _NC + cid, cid, sid


def _zero_fill(buf, nrows):
    z = jnp.zeros((16,), _f32)
    nseg = buf.shape[1] // 16

    def body(i, _):
        for q in range(nseg):
            buf[i, pl.ds(16 * q, 16)] = z
        return 0

    lax.fori_loop(0, nrows, body, 0)


def _zero_acc(zbuf, acc, sid):
    pltpu.sync_copy(zbuf.at[pl.ds(0, _RPS)], acc.at[pl.ds(sid * _RPS, _RPS)])

    @pl.when(sid == _NS - 1)
    def _():
        pltpu.sync_copy(
            zbuf.at[pl.ds(0, _TAIL)], acc.at[pl.ds(_NS * _RPS, _TAIL)]
        )


def _copy_out(acc, out, cid, sid):
    pltpu.sync_copy(
        acc.at[pl.ds(sid * _RPS, _RPS)], out.at[cid, pl.ds(sid * _RPS, _RPS)]
    )

    @pl.when(sid == _NS - 1)
    def _():
        pltpu.sync_copy(
            acc.at[pl.ds(_NS * _RPS, _TAIL)],
            out.at[cid, pl.ds(_NS * _RPS, _TAIL)],
        )


def _body_a(rowm, colm, trow, tcol, htab, e_out, pa_out, pb_out,
            irow, icol, grow, gcol, ghw, vrow, ebuf, ones, zbuf,
            acc_row, acc_b, sem1, sem2, sem3):
    wid, cid, sid = _wid_cid_sid()
    lanes = lax.iota(jnp.int32, 16)
    m8 = lanes < 8

    def initones(i, _):
        ones[i, :] = jnp.where(lanes == 0, 1.0, 0.0).astype(_f32)
        return 0

    lax.fori_loop(0, _CB, initones, 0)
    _zero_fill(zbuf, _RPS)
    _zero_acc(zbuf, acc_row, sid)
    _zero_acc(zbuf, acc_b, sid)
    plsc.subcore_barrier()

    pltpu.sync_copy(rowm.at[wid], irow)
    pltpu.sync_copy(colm.at[wid], icol)

    def chunk(j, _):
        ir = irow.at[j]
        ic = icol.at[j]
        d1 = pltpu.async_copy(trow.at[ir], grow, sem1)
        d2 = pltpu.async_copy(tcol.at[ic], gcol, sem2)
        d3 = pltpu.async_copy(htab.at[ic], ghw, sem3)
        d1.wait()
        d2.wait()
        d3.wait()

        def edge(t, _):
            a = grow[t, :] + gcol[t, :]
            ex = jnp.exp(jnp.maximum(a, a * 0.2))
            ebuf[t, :] = ex
            vrow[t, :] = jnp.where(m8, ex, ghw[t, :])
            return 0

        lax.fori_loop(0, _CB, edge, 0)
        pltpu.sync_copy(vrow, acc_row.at[ir], add=True)
        pltpu.sync_copy(ones, acc_b.at[ic], add=True)
        pltpu.sync_copy(ebuf, e_out.at[pl.ds(wid * _EPW + j * _CB, _CB)])
        return 0

    lax.fori_loop(0, _NCH, chunk, 0)
    plsc.subcore_barrier()
    _copy_out(acc_row, pa_out, cid, sid)
    _copy_out(acc_b, pb_out, cid, sid)


_sc_a = pl.kernel(
    _body_a,
    out_type=(
        jax.ShapeDtypeStruct((_K, 16), _f32),
        jax.ShapeDtypeStruct((_NC, _N, 16), _f32),
        jax.ShapeDtypeStruct((_NC, _N, 16), _f32),
    ),
    mesh=_mesh,
    compiler_params=pltpu.CompilerParams(use_tc_tiling_on_sc=False),
    scratch_types=[
        pltpu.VMEM((_NCH, _CB), jnp.int32),
        pltpu.VMEM((_NCH, _CB), jnp.int32),
        pltpu.VMEM((_CB, 16), _f32),
        pltpu.VMEM((_CB, 16), _f32),
        pltpu.VMEM((_CB, 16), _f32),
        pltpu.VMEM((_CB, 16), _f32),
        pltpu.VMEM((_CB, 16), _f32),
        pltpu.VMEM((_CB, 16), _f32),
        pltpu.VMEM((_RPS, 16), _f32),
        pltpu.VMEM_SHARED((_N, 16), _f32),
        pltpu.VMEM_SHARED((_N, 16), _f32),
        pltpu.SemaphoreType.DMA,
        pltpu.SemaphoreType.DMA,
        pltpu.SemaphoreType.DMA,
    ],
)


def _body_b(rowm, colm, rtab, e_hbm, al_out, po_out,
            irow, icol, gr, ech, abuf, vbuf, zbuf, acc, sem1, sem2):
    wid, cid, sid = _wid_cid_sid()

    _zero_fill(zbuf, _RPS)
    _zero_acc(zbuf, acc, sid)
    plsc.subcore_barrier()

    pltpu.sync_copy(rowm.at[wid], irow)
    pltpu.sync_copy(colm.at[wid], icol)

    def chunk(j, _):
        base = wid * _EPW + j * _CB
        d1 = pltpu.async_copy(rtab.at[irow.at[j]], gr, sem1)
        d2 = pltpu.async_copy(e_hbm.at[pl.ds(base, _CB)], ech, sem2)
        d1.wait()
        d2.wait()

        def edge(t, _):
            al = ech[t, :] * gr[t, pl.ds(64, 16)]
            abuf[t, :] = al
            for q in range(4):
                vbuf[t, pl.ds(16 * q, 16)] = gr[t, pl.ds(16 * q, 16)] * al
            return 0

        lax.fori_loop(0, _CB, edge, 0)
        pltpu.sync_copy(vbuf, acc.at[icol.at[j]], add=True)
        pltpu.sync_copy(abuf, al_out.at[pl.ds(base, _CB)])
        return 0

    lax.fori_loop(0, _NCH, chunk, 0)
    plsc.subcore_barrier()
    _copy_out(acc, po_out, cid, sid)


_sc_b = pl.kernel(
    _body_b,
    out_type=(
        jax.ShapeDtypeStruct((_K, 16), _f32),
        jax.ShapeDtypeStruct((_NC, _N, 64), _f32),
    ),
    mesh=_mesh,
    compiler_params=pltpu.CompilerParams(use_tc_tiling_on_sc=False),
    scratch_types=[
        pltpu.VMEM((_NCH, _CB), jnp.int32),
        pltpu.VMEM((_NCH, _CB), jnp.int32),
        pltpu.VMEM((_CB, 80), _f32),
        pltpu.VMEM((_CB, 16), _f32),
        pltpu.VMEM((_CB, 16), _f32),
        pltpu.VMEM((_CB, 64), _f32),
        pltpu.VMEM((_RPS, 64), _f32),
        pltpu.VMEM_SHARED((_N, 64), _f32),
        pltpu.SemaphoreType.DMA,
        pltpu.SemaphoreType.DMA,
    ],
)


def _body_c(rowm, colm, oetab, al_hbm, pc_out,
            irow, icol, go, ach, vbuf, zbuf, acc, sem1, sem2):
    wid, cid, sid = _wid_cid_sid()

    _zero_fill(zbuf, _RPS)
    _zero_acc(zbuf, acc, sid)
    plsc.subcore_barrier()

    pltpu.sync_copy(rowm.at[wid], irow)
    pltpu.sync_copy(colm.at[wid], icol)

    def chunk(j, _):
        base = wid * _EPW + j * _CB
        d1 = pltpu.async_copy(oetab.at[icol.at[j]], go, sem1)
        d2 = pltpu.async_copy(al_hbm.at[pl.ds(base, _CB)], ach, sem2)
        d1.wait()
        d2.wait()

        def edge(t, _):
            al = ach[t, :]
            for q in range(4):
                vbuf[t, pl.ds(16 * q, 16)] = go[t, pl.ds(16 * q, 16)] * al
            return 0

        lax.fori_loop(0, _CB, edge, 0)
        pltpu.sync_copy(vbuf, acc.at[irow.at[j]], add=True)
        return 0

    lax.fori_loop(0, _NCH, chunk, 0)
    plsc.subcore_barrier()
    _copy_out(acc, pc_out, cid, sid)


_sc_c = pl.kernel(
    _body_c,
    out_type=jax.ShapeDtypeStruct((_NC, _N, 64), _f32),
    mesh=_mesh,
    compiler_params=pltpu.CompilerParams(use_tc_tiling_on_sc=False),
    scratch_types=[
        pltpu.VMEM((_NCH, _CB), jnp.int32),
        pltpu.VMEM((_NCH, _CB), jnp.int32),
        pltpu.VMEM((_CB, 64), _f32),
        pltpu.VMEM((_CB, 16), _f32),
        pltpu.VMEM((_CB, 64), _f32),
        pltpu.VMEM((_RPS, 64), _f32),
        pltpu.VMEM_SHARED((_N, 64), _f32),
        pltpu.SemaphoreType.DMA,
        pltpu.SemaphoreType.DMA,
    ],
)


def _body_g(gm, sm, tab, p_out, gib, sib, gbuf, zbuf, acc, sem1):
    wid, cid, sid = _wid_cid_sid()
    _zero_fill(zbuf, _RPS)
    _zero_acc(zbuf, acc, sid)
    plsc.subcore_barrier()

    pltpu.sync_copy(gm.at[wid], gib)
    pltpu.sync_copy(sm.at[wid], sib)

    def chunk(j, _):
        pltpu.async_copy(tab.at[gib.at[j]], gbuf, sem1).wait()
        pltpu.sync_copy(gbuf, acc.at[sib.at[j]], add=True)
        return 0

    lax.fori_loop(0, _NCH, chunk, 0)
    plsc.subcore_barrier()
    _copy_out(acc, p_out, cid, sid)


_sc_g = pl.kernel(
    _body_g,
    out_type=jax.ShapeDtypeStruct((_NC, _N, 16), _f32),
    mesh=_mesh,
    compiler_params=pltpu.CompilerParams(use_tc_tiling_on_sc=False),
    scratch_types=[
        pltpu.VMEM((_NCH, _CB), jnp.int32),
        pltpu.VMEM((_NCH, _CB), jnp.int32),
        pltpu.VMEM((_CB, 16), _f32),
        pltpu.VMEM((_RPS, 16), _f32),
        pltpu.VMEM_SHARED((_N, 16), _f32),
        pltpu.SemaphoreType.DMA,
    ],
)


def _tc1(x_ref, w1_ref, ai_ref, aj_ref, hw_ref, xw_ref, trow_ref, tcol_ref,
         htab_ref):
    xw = jnp.dot(x_ref[...], w1_ref[...], preferred_element_type=_f32)
    xw_ref[...] = xw
    si = jnp.dot(xw, ai_ref[...], preferred_element_type=_f32)
    sj = jnp.dot(xw, aj_ref[...], preferred_element_type=_f32)
    trow_ref[...] = jnp.concatenate([si, si], axis=1)
    tcol_ref[...] = jnp.concatenate([sj, sj], axis=1)
    htab_ref[...] = jnp.concatenate(
        [jnp.zeros((_N, 8), _f32), hw_ref[...], jnp.zeros((_N, 7), _f32)],
        axis=1,
    )


def _tc2(pa_ref, pb_ref, xw_ref, r_ref, db_ref):
    sa = pa_ref[0] + pa_ref[1]
    ssum = sa[:, :8]
    d = sa[:, 8:9]
    b = pb_ref[0][:, 0:1] + pb_ref[1][:, 0:1]
    inv = 1.0 / (ssum + 1e-16)
    dinv = jnp.where(d > 0, 1.0 / jnp.where(d > 0, d, 1.0), 0.0)
    binv = jnp.where(b > 0, 1.0 / jnp.where(b > 0, b, 1.0), 0.0)
    r_ref[...] = jnp.concatenate([xw_ref[...], inv, inv], axis=1)
    db_ref[...] = jnp.concatenate(
        [dinv, binv, jnp.zeros((_N, 6), _f32)], axis=1
    )


def _tc3(po_ref, db_ref, oe_ref):
    oe_ref[...] = (po_ref[0] + po_ref[1]) * db_ref[...][:, 1:2]


def _tc4(pc_ref, db_ref, b1_ref, w2_ref, h2_ref):
    s = (pc_ref[0] + pc_ref[1]) * db_ref[...][:, 0:1] + b1_ref[...]
    h = jnp.where(s > 0, s, jnp.exp(jnp.minimum(s, 0.0)) - 1.0)
    hh = jnp.dot(h, w2_ref[...], preferred_element_type=_f32)
    h2_ref[...] = jnp.concatenate(
        [hh, jnp.zeros((_N, 16 - _NCLS), _f32)], axis=1
    )


def _tc5(pd_ref, db_ref, oe2_ref):
    oe2_ref[...] = (pd_ref[0] + pd_ref[1]) * db_ref[...][:, 1:2]


def _tc6(pe_ref, db_ref, b2_ref, out_ref):
    s = (pe_ref[0] + pe_ref[1])[:, :_NCLS] * db_ref[...][:, 0:1] + b2_ref[...]
    m = jnp.max(s, axis=1, keepdims=True)
    zz = s - m
    out_ref[...] = zz - jnp.log(jnp.sum(jnp.exp(zz), axis=1, keepdims=True))


def _call_tc(fn, out_shape, *args):
    return pl.pallas_call(fn, out_shape=out_shape)(*args)


def kernel(x, edge_index, hyperedge_weight, W1, att1, b1, W2, b2):
    rowm = edge_index[0].reshape(_NW, _NCH, _CB)
    colm = edge_index[1].reshape(_NW, _NCH, _CB)
    hw = hyperedge_weight.reshape(_N, 1)
    perm = jnp.arange(64).reshape(_HEADS, _HID).T.reshape(64)
    w1p = W1[:, perm]
    b1p = b1[perm]
    w2p = W2[perm, :]
    eye = jnp.eye(_HEADS, dtype=_f32)
    a_i = (att1[0, :, :_HID].T[:, :, None] * eye[None, :, :]).reshape(64, 8)
    a_j = (att1[0, :, _HID:].T[:, :, None] * eye[None, :, :]).reshape(64, 8)

    xw, trow, tcol, htab = _call_tc(
        _tc1,
        (
            jax.ShapeDtypeStruct((_N, 64), _f32),
            jax.ShapeDtypeStruct((_N, 16), _f32),
            jax.ShapeDtypeStruct((_N, 16), _f32),
            jax.ShapeDtypeStruct((_N, 16), _f32),
        ),
        x, w1p, a_i, a_j, hw,
    )

    e, pa, pb = _sc_a(rowm, colm, trow, tcol, htab)

    rtab, db = _call_tc(
        _tc2,
        (
            jax.ShapeDtypeStruct((_N, 80), _f32),
            jax.ShapeDtypeStruct((_N, 8), _f32),
        ),
        pa, pb, xw,
    )

    alpha, po = _sc_b(rowm, colm, rtab, e)

    oe = _call_tc(_tc3, jax.ShapeDtypeStruct((_N, 64), _f32), po, db)

    pc = _sc_c(rowm, colm, oe, alpha)

    h2 = _call_tc(
        _tc4, jax.ShapeDtypeStruct((_N, 16), _f32),
        pc, db, b1p.reshape(1, -1), w2p,
    )

    pd = _sc_g(rowm, colm, h2)

    oe2 = _call_tc(_tc5, jax.ShapeDtypeStruct((_N, 16), _f32), pd, db)

    pe = _sc_g(colm, rowm, oe2)

    out = _call_tc(
        _tc6, jax.ShapeDtypeStruct((_N, _NCLS), _f32),
        pe, db, b2.reshape(1, -1),
    )
    return out


# speedup vs baseline: 92.5438x; 1.2605x over previous
import jax
import jax.numpy as jnp
from jax import lax
from jax.experimental import pallas as pl
from jax.experimental.pallas import tpu as pltpu
from jax.experimental.pallas import tpu_sc as plsc

_N = 10000
_K = 320000
_HEADS = 8
_HID = 8
_NCLS = 7

_NC = 2
_NS = 16
_NW = _NC * _NS
_EPW = _K // _NW
_CB = 80
_NCH = _EPW // _CB
_RPS = 624
_TAIL = _N - _NS * _RPS

_mesh = plsc.VectorSubcoreMesh(
    core_axis_name="c", subcore_axis_name="s", num_cores=_NC, num_subcores=_NS
)

_f32 = jnp.float32


def _wid_cid_sid():
    cid = lax.axis_index("c")
    sid = lax.axis_index("s")
    return sid * _NC + cid, cid, sid


def _zero_fill(buf, nrows):
    z = jnp.zeros((16,), _f32)
    nseg = buf.shape[1] // 16

    def body(i, _):
        for q in range(nseg):
            buf[i, pl.ds(16 * q, 16)] = z
        return 0

    lax.fori_loop(0, nrows, body, 0)


def _zero_acc(zbuf, acc, sid):
    pltpu.sync_copy(zbuf.at[pl.ds(0, _RPS)], acc.at[pl.ds(sid * _RPS, _RPS)])

    @pl.when(sid == _NS - 1)
    def _():
        pltpu.sync_copy(
            zbuf.at[pl.ds(0, _TAIL)], acc.at[pl.ds(_NS * _RPS, _TAIL)]
        )


def _copy_out(acc, out, cid, sid):
    pltpu.sync_copy(
        acc.at[pl.ds(sid * _RPS, _RPS)], out.at[cid, pl.ds(sid * _RPS, _RPS)]
    )

    @pl.when(sid == _NS - 1)
    def _():
        pltpu.sync_copy(
            acc.at[pl.ds(_NS * _RPS, _TAIL)],
            out.at[cid, pl.ds(_NS * _RPS, _TAIL)],
        )


def _body_a(rowm, colm, trow, tcol, htab, e_out, pa_out, pb_out,
            irow, icol, grow, gcol, ghw, vrow, ebuf, ones, zbuf,
            acc_row, acc_b, sem1, sem2, sem3):
    wid, cid, sid = _wid_cid_sid()
    lanes = lax.iota(jnp.int32, 16)
    m8 = lanes < 8

    def initones(i, _):
        ones[i, :] = jnp.where(lanes == 0, 1.0, 0.0).astype(_f32)
        return 0

    lax.fori_loop(0, _CB, initones, 0)
    _zero_fill(zbuf, _RPS)
    _zero_acc(zbuf, acc_row, sid)
    _zero_acc(zbuf, acc_b, sid)
    plsc.subcore_barrier()

    pltpu.sync_copy(rowm.at[wid], irow)
    pltpu.sync_copy(colm.at[wid], icol)

    def chunk(j, _):
        ir = irow.at[j]
        ic = icol.at[j]
        d1 = pltpu.async_copy(trow.at[ir], grow, sem1)
        d2 = pltpu.async_copy(tcol.at[ic], gcol, sem2)
        d3 = pltpu.async_copy(htab.at[ic], ghw, sem3)
        d1.wait()
        d2.wait()
        d3.wait()

        @plsc.parallel_loop(0, _CB, unroll=8)
        def edge(t):
            a = grow[t, :] + gcol[t, :]
            ex = jnp.exp(jnp.maximum(a, a * 0.2))
            ebuf[t, :] = ex
            vrow[t, :] = jnp.where(m8, ex, ghw[t, :])
        pltpu.sync_copy(vrow, acc_row.at[ir], add=True)
        pltpu.sync_copy(ones, acc_b.at[ic], add=True)
        pltpu.sync_copy(ebuf, e_out.at[pl.ds(wid * _EPW + j * _CB, _CB)])
        return 0

    lax.fori_loop(0, _NCH, chunk, 0)
    plsc.subcore_barrier()
    _copy_out(acc_row, pa_out, cid, sid)
    _copy_out(acc_b, pb_out, cid, sid)


_sc_a = pl.kernel(
    _body_a,
    out_type=(
        jax.ShapeDtypeStruct((_K, 16), _f32),
        jax.ShapeDtypeStruct((_NC, _N, 16), _f32),
        jax.ShapeDtypeStruct((_NC, _N, 16), _f32),
    ),
    mesh=_mesh,
    compiler_params=pltpu.CompilerParams(use_tc_tiling_on_sc=False),
    scratch_types=[
        pltpu.VMEM((_NCH, _CB), jnp.int32),
        pltpu.VMEM((_NCH, _CB), jnp.int32),
        pltpu.VMEM((_CB, 16), _f32),
        pltpu.VMEM((_CB, 16), _f32),
        pltpu.VMEM((_CB, 16), _f32),
        pltpu.VMEM((_CB, 16), _f32),
        pltpu.VMEM((_CB, 16), _f32),
        pltpu.VMEM((_CB, 16), _f32),
        pltpu.VMEM((_RPS, 16), _f32),
        pltpu.VMEM_SHARED((_N, 16), _f32),
        pltpu.VMEM_SHARED((_N, 16), _f32),
        pltpu.SemaphoreType.DMA,
        pltpu.SemaphoreType.DMA,
        pltpu.SemaphoreType.DMA,
    ],
)


def _body_b(rowm, colm, rtab, e_hbm, al_out, po_out,
            irow, icol, gr, ech, abuf, vbuf, zbuf, acc, sem1, sem2):
    wid, cid, sid = _wid_cid_sid()

    _zero_fill(zbuf, _RPS)
    _zero_acc(zbuf, acc, sid)
    plsc.subcore_barrier()

    pltpu.sync_copy(rowm.at[wid], irow)
    pltpu.sync_copy(colm.at[wid], icol)

    def chunk(j, _):
        base = wid * _EPW + j * _CB
        d1 = pltpu.async_copy(rtab.at[irow.at[j]], gr, sem1)
        d2 = pltpu.async_copy(e_hbm.at[pl.ds(base, _CB)], ech, sem2)
        d1.wait()
        d2.wait()

        @plsc.parallel_loop(0, _CB, unroll=8)
        def edge(t):
            al = ech[t, :] * gr[t, pl.ds(64, 16)]
            abuf[t, :] = al
            for q in range(4):
                vbuf[t, pl.ds(16 * q, 16)] = gr[t, pl.ds(16 * q, 16)] * al
        pltpu.sync_copy(vbuf, acc.at[icol.at[j]], add=True)
        pltpu.sync_copy(abuf, al_out.at[pl.ds(base, _CB)])
        return 0

    lax.fori_loop(0, _NCH, chunk, 0)
    plsc.subcore_barrier()
    _copy_out(acc, po_out, cid, sid)


_sc_b = pl.kernel(
    _body_b,
    out_type=(
        jax.ShapeDtypeStruct((_K, 16), _f32),
        jax.ShapeDtypeStruct((_NC, _N, 64), _f32),
    ),
    mesh=_mesh,
    compiler_params=pltpu.CompilerParams(use_tc_tiling_on_sc=False),
    scratch_types=[
        pltpu.VMEM((_NCH, _CB), jnp.int32),
        pltpu.VMEM((_NCH, _CB), jnp.int32),
        pltpu.VMEM((_CB, 80), _f32),
        pltpu.VMEM((_CB, 16), _f32),
        pltpu.VMEM((_CB, 16), _f32),
        pltpu.VMEM((_CB, 64), _f32),
        pltpu.VMEM((_RPS, 64), _f32),
        pltpu.VMEM_SHARED((_N, 64), _f32),
        pltpu.SemaphoreType.DMA,
        pltpu.SemaphoreType.DMA,
    ],
)


def _body_c(rowm, colm, oetab, al_hbm, pc_out,
            irow, icol, go, ach, vbuf, zbuf, acc, sem1, sem2):
    wid, cid, sid = _wid_cid_sid()

    _zero_fill(zbuf, _RPS)
    _zero_acc(zbuf, acc, sid)
    plsc.subcore_barrier()

    pltpu.sync_copy(rowm.at[wid], irow)
    pltpu.sync_copy(colm.at[wid], icol)

    def chunk(j, _):
        base = wid * _EPW + j * _CB
        d1 = pltpu.async_copy(oetab.at[icol.at[j]], go, sem1)
        d2 = pltpu.async_copy(al_hbm.at[pl.ds(base, _CB)], ach, sem2)
        d1.wait()
        d2.wait()

        @plsc.parallel_loop(0, _CB, unroll=8)
        def edge(t):
            al = ach[t, :]
            for q in range(4):
                vbuf[t, pl.ds(16 * q, 16)] = go[t, pl.ds(16 * q, 16)] * al
        pltpu.sync_copy(vbuf, acc.at[irow.at[j]], add=True)
        return 0

    lax.fori_loop(0, _NCH, chunk, 0)
    plsc.subcore_barrier()
    _copy_out(acc, pc_out, cid, sid)


_sc_c = pl.kernel(
    _body_c,
    out_type=jax.ShapeDtypeStruct((_NC, _N, 64), _f32),
    mesh=_mesh,
    compiler_params=pltpu.CompilerParams(use_tc_tiling_on_sc=False),
    scratch_types=[
        pltpu.VMEM((_NCH, _CB), jnp.int32),
        pltpu.VMEM((_NCH, _CB), jnp.int32),
        pltpu.VMEM((_CB, 64), _f32),
        pltpu.VMEM((_CB, 16), _f32),
        pltpu.VMEM((_CB, 64), _f32),
        pltpu.VMEM((_RPS, 64), _f32),
        pltpu.VMEM_SHARED((_N, 64), _f32),
        pltpu.SemaphoreType.DMA,
        pltpu.SemaphoreType.DMA,
    ],
)


def _body_g(gm, sm, tab, p_out, gib, sib, gbuf, zbuf, acc, sem1):
    wid, cid, sid = _wid_cid_sid()
    _zero_fill(zbuf, _RPS)
    _zero_acc(zbuf, acc, sid)
    plsc.subcore_barrier()

    pltpu.sync_copy(gm.at[wid], gib)
    pltpu.sync_copy(sm.at[wid], sib)

    def chunk(j, _):
        pltpu.async_copy(tab.at[gib.at[j]], gbuf, sem1).wait()
        pltpu.sync_copy(gbuf, acc.at[sib.at[j]], add=True)
        return 0

    lax.fori_loop(0, _NCH, chunk, 0)
    plsc.subcore_barrier()
    _copy_out(acc, p_out, cid, sid)


_sc_g = pl.kernel(
    _body_g,
    out_type=jax.ShapeDtypeStruct((_NC, _N, 16), _f32),
    mesh=_mesh,
    compiler_params=pltpu.CompilerParams(use_tc_tiling_on_sc=False),
    scratch_types=[
        pltpu.VMEM((_NCH, _CB), jnp.int32),
        pltpu.VMEM((_NCH, _CB), jnp.int32),
        pltpu.VMEM((_CB, 16), _f32),
        pltpu.VMEM((_RPS, 16), _f32),
        pltpu.VMEM_SHARED((_N, 16), _f32),
        pltpu.SemaphoreType.DMA,
    ],
)


def _tc1(x_ref, w1_ref, ai_ref, aj_ref, hw_ref, xw_ref, trow_ref, tcol_ref,
         htab_ref):
    xw = jnp.dot(x_ref[...], w1_ref[...], preferred_element_type=_f32)
    xw_ref[...] = xw
    si = jnp.dot(xw, ai_ref[...], preferred_element_type=_f32)
    sj = jnp.dot(xw, aj_ref[...], preferred_element_type=_f32)
    trow_ref[...] = jnp.concatenate([si, si], axis=1)
    tcol_ref[...] = jnp.concatenate([sj, sj], axis=1)
    htab_ref[...] = jnp.concatenate(
        [jnp.zeros((_N, 8), _f32), hw_ref[...], jnp.zeros((_N, 7), _f32)],
        axis=1,
    )


def _tc2(pa_ref, pb_ref, xw_ref, r_ref, db_ref):
    sa = pa_ref[0] + pa_ref[1]
    ssum = sa[:, :8]
    d = sa[:, 8:9]
    b = pb_ref[0][:, 0:1] + pb_ref[1][:, 0:1]
    inv = 1.0 / (ssum + 1e-16)
    dinv = jnp.where(d > 0, 1.0 / jnp.where(d > 0, d, 1.0), 0.0)
    binv = jnp.where(b > 0, 1.0 / jnp.where(b > 0, b, 1.0), 0.0)
    r_ref[...] = jnp.concatenate([xw_ref[...], inv, inv], axis=1)
    db_ref[...] = jnp.concatenate(
        [dinv, binv, jnp.zeros((_N, 6), _f32)], axis=1
    )


def _tc3(po_ref, db_ref, oe_ref):
    oe_ref[...] = (po_ref[0] + po_ref[1]) * db_ref[...][:, 1:2]


def _tc4(pc_ref, db_ref, b1_ref, w2_ref, h2_ref):
    s = (pc_ref[0] + pc_ref[1]) * db_ref[...][:, 0:1] + b1_ref[...]
    h = jnp.where(s > 0, s, jnp.exp(jnp.minimum(s, 0.0)) - 1.0)
    hh = jnp.dot(h, w2_ref[...], preferred_element_type=_f32)
    h2_ref[...] = jnp.concatenate(
        [hh, jnp.zeros((_N, 16 - _NCLS), _f32)], axis=1
    )


def _tc5(pd_ref, db_ref, oe2_ref):
    oe2_ref[...] = (pd_ref[0] + pd_ref[1]) * db_ref[...][:, 1:2]


def _tc6(pe_ref, db_ref, b2_ref, out_ref):
    s = (pe_ref[0] + pe_ref[1])[:, :_NCLS] * db_ref[...][:, 0:1] + b2_ref[...]
    m = jnp.max(s, axis=1, keepdims=True)
    zz = s - m
    out_ref[...] = zz - jnp.log(jnp.sum(jnp.exp(zz), axis=1, keepdims=True))


def _call_tc(fn, out_shape, *args):
    return pl.pallas_call(fn, out_shape=out_shape)(*args)


def kernel(x, edge_index, hyperedge_weight, W1, att1, b1, W2, b2):
    rowm = edge_index[0].reshape(_NW, _NCH, _CB)
    colm = edge_index[1].reshape(_NW, _NCH, _CB)
    hw = hyperedge_weight.reshape(_N, 1)
    perm = jnp.arange(64).reshape(_HEADS, _HID).T.reshape(64)
    w1p = W1[:, perm]
    b1p = b1[perm]
    w2p = W2[perm, :]
    eye = jnp.eye(_HEADS, dtype=_f32)
    a_i = (att1[0, :, :_HID].T[:, :, None] * eye[None, :, :]).reshape(64, 8)
    a_j = (att1[0, :, _HID:].T[:, :, None] * eye[None, :, :]).reshape(64, 8)

    xw, trow, tcol, htab = _call_tc(
        _tc1,
        (
            jax.ShapeDtypeStruct((_N, 64), _f32),
            jax.ShapeDtypeStruct((_N, 16), _f32),
            jax.ShapeDtypeStruct((_N, 16), _f32),
            jax.ShapeDtypeStruct((_N, 16), _f32),
        ),
        x, w1p, a_i, a_j, hw,
    )

    e, pa, pb = _sc_a(rowm, colm, trow, tcol, htab)

    rtab, db = _call_tc(
        _tc2,
        (
            jax.ShapeDtypeStruct((_N, 80), _f32),
            jax.ShapeDtypeStruct((_N, 8), _f32),
        ),
        pa, pb, xw,
    )

    alpha, po = _sc_b(rowm, colm, rtab, e)

    oe = _call_tc(_tc3, jax.ShapeDtypeStruct((_N, 64), _f32), po, db)

    pc = _sc_c(rowm, colm, oe, alpha)

    h2 = _call_tc(
        _tc4, jax.ShapeDtypeStruct((_N, 16), _f32),
        pc, db, b1p.reshape(1, -1), w2p,
    )

    pd = _sc_g(rowm, colm, h2)

    oe2 = _call_tc(_tc5, jax.ShapeDtypeStruct((_N, 16), _f32), pd, db)

    pe = _sc_g(colm, rowm, oe2)

    out = _call_tc(
        _tc6, jax.ShapeDtypeStruct((_N, _NCLS), _f32),
        pe, db, b2.reshape(1, -1),
    )
    return out


# speedup vs baseline: 142.1375x; 1.5359x over previous
import jax
import jax.numpy as jnp
from jax import lax
from jax.experimental import pallas as pl
from jax.experimental.pallas import tpu as pltpu
from jax.experimental.pallas import tpu_sc as plsc

_N = 10000
_K = 320000
_HEADS = 8
_HID = 8
_NCLS = 7

_NC = 2
_NS = 16
_NW = _NC * _NS
_EPW = _K // _NW
_CB = 80
_NCH = _EPW // _CB
_RPS = 624
_TAIL = _N - _NS * _RPS

_mesh = plsc.VectorSubcoreMesh(
    core_axis_name="c", subcore_axis_name="s", num_cores=_NC, num_subcores=_NS
)

_f32 = jnp.float32


def _wid_cid_sid():
    cid = lax.axis_index("c")
    sid = lax.axis_index("s")
    return sid * _NC + cid, cid, sid


def _zero_fill(buf, nrows):
    z = jnp.zeros((16,), _f32)
    nseg = buf.shape[1] // 16

    def body(i, _):
        for q in range(nseg):
            buf[i, pl.ds(16 * q, 16)] = z
        return 0

    lax.fori_loop(0, nrows, body, 0)


def _zero_acc(zbuf, acc, sid):
    pltpu.sync_copy(zbuf.at[pl.ds(0, _RPS)], acc.at[pl.ds(sid * _RPS, _RPS)])

    @pl.when(sid == _NS - 1)
    def _():
        pltpu.sync_copy(
            zbuf.at[pl.ds(0, _TAIL)], acc.at[pl.ds(_NS * _RPS, _TAIL)]
        )


def _copy_out(acc, out, cid, sid):
    pltpu.sync_copy(
        acc.at[pl.ds(sid * _RPS, _RPS)], out.at[cid, pl.ds(sid * _RPS, _RPS)]
    )

    @pl.when(sid == _NS - 1)
    def _():
        pltpu.sync_copy(
            acc.at[pl.ds(_NS * _RPS, _TAIL)],
            out.at[cid, pl.ds(_NS * _RPS, _TAIL)],
        )


def _run_ring(do_chunk):
    def pair(jj, _):
        for b in range(2):
            do_chunk(2 * jj + b, b)
        return 0

    lax.fori_loop(0, _NCH // 2, pair, 0)
    do_chunk(_NCH - 1, 0)


def _body_a(rowm, colm, trow, tcol, htab, e_out, pa_out, pb_out,
            irow, icol, grow0, grow1, gcol0, gcol1, ghw0, ghw1,
            vrow, ebuf0, ebuf1, ones, zbuf, acc_row, acc_b,
            sg0, sg1, st0, st1):
    wid, cid, sid = _wid_cid_sid()
    lanes = lax.iota(jnp.int32, 16)
    m8 = lanes < 8
    grows = (grow0, grow1)
    gcols = (gcol0, gcol1)
    ghws = (ghw0, ghw1)
    ebufs = (ebuf0, ebuf1)
    sgs = (sg0, sg1)
    sts = (st0, st1)

    def initones(i, _):
        ones[i, :] = jnp.where(lanes == 0, 1.0, 0.0).astype(_f32)
        return 0

    lax.fori_loop(0, _CB, initones, 0)
    _zero_fill(zbuf, _RPS)
    _zero_acc(zbuf, acc_row, sid)
    _zero_acc(zbuf, acc_b, sid)
    plsc.subcore_barrier()

    pltpu.sync_copy(rowm.at[wid], irow)
    pltpu.sync_copy(colm.at[wid], icol)

    def g_descs(j, b):
        return (
            pltpu.make_async_copy(trow.at[irow.at[j]], grows[b], sgs[b]),
            pltpu.make_async_copy(tcol.at[icol.at[j]], gcols[b], sgs[b]),
            pltpu.make_async_copy(htab.at[icol.at[j]], ghws[b], sgs[b]),
        )

    def e_desc(j, b):
        base = wid * _EPW + j * _CB
        return pltpu.make_async_copy(
            ebufs[b], e_out.at[pl.ds(base, _CB)], sts[b]
        )

    for d in g_descs(0, 0):
        d.start()

    def do_chunk(j, b):
        @pl.when(j + 1 < _NCH)
        def _():
            for d in g_descs(j + 1, 1 - b):
                d.start()

        @pl.when(j >= 2)
        def _():
            e_desc(j - 2, b).wait()

        for d in g_descs(j, b):
            d.wait()
        grow, gcol, ghw, ebuf = grows[b], gcols[b], ghws[b], ebufs[b]

        @plsc.parallel_loop(0, _CB, unroll=8)
        def edge(t):
            a = grow[t, :] + gcol[t, :]
            ex = jnp.exp(jnp.maximum(a, a * 0.2))
            ebuf[t, :] = ex
            vrow[t, :] = jnp.where(m8, ex, ghw[t, :])

        pltpu.sync_copy(vrow, acc_row.at[irow.at[j]], add=True)
        pltpu.sync_copy(ones, acc_b.at[icol.at[j]], add=True)
        e_desc(j, b).start()

    _run_ring(do_chunk)
    e_desc(_NCH - 2, 1).wait()
    e_desc(_NCH - 1, 0).wait()
    plsc.subcore_barrier()
    _copy_out(acc_row, pa_out, cid, sid)
    _copy_out(acc_b, pb_out, cid, sid)


_sc_a = pl.kernel(
    _body_a,
    out_type=(
        jax.ShapeDtypeStruct((_K, 16), _f32),
        jax.ShapeDtypeStruct((_NC, _N, 16), _f32),
        jax.ShapeDtypeStruct((_NC, _N, 16), _f32),
    ),
    mesh=_mesh,
    compiler_params=pltpu.CompilerParams(use_tc_tiling_on_sc=False),
    scratch_types=[
        pltpu.VMEM((_NCH, _CB), jnp.int32),
        pltpu.VMEM((_NCH, _CB), jnp.int32),
        pltpu.VMEM((_CB, 16), _f32),
        pltpu.VMEM((_CB, 16), _f32),
        pltpu.VMEM((_CB, 16), _f32),
        pltpu.VMEM((_CB, 16), _f32),
        pltpu.VMEM((_CB, 16), _f32),
        pltpu.VMEM((_CB, 16), _f32),
        pltpu.VMEM((_CB, 16), _f32),
        pltpu.VMEM((_CB, 16), _f32),
        pltpu.VMEM((_CB, 16), _f32),
        pltpu.VMEM((_CB, 16), _f32),
        pltpu.VMEM((_RPS, 16), _f32),
        pltpu.VMEM_SHARED((_N, 16), _f32),
        pltpu.VMEM_SHARED((_N, 16), _f32),
        pltpu.SemaphoreType.DMA,
        pltpu.SemaphoreType.DMA,
        pltpu.SemaphoreType.DMA,
        pltpu.SemaphoreType.DMA,
    ],
)


def _body_b(rowm, colm, rtab, e_hbm, al_out, po_out,
            irow, icol, gr0, gr1, ech0, ech1, abuf0, abuf1, vbuf, zbuf, acc,
            sg0, sg1, st0, st1):
    wid, cid, sid = _wid_cid_sid()
    grs = (gr0, gr1)
    echs = (ech0, ech1)
    abufs = (abuf0, abuf1)
    sgs = (sg0, sg1)
    sts = (st0, st1)

    _zero_fill(zbuf, _RPS)
    _zero_acc(zbuf, acc, sid)
    plsc.subcore_barrier()

    pltpu.sync_copy(rowm.at[wid], irow)
    pltpu.sync_copy(colm.at[wid], icol)

    def g_descs(j, b):
        base = wid * _EPW + j * _CB
        return (
            pltpu.make_async_copy(rtab.at[irow.at[j]], grs[b], sgs[b]),
            pltpu.make_async_copy(
                e_hbm.at[pl.ds(base, _CB)], echs[b], sgs[b]
            ),
        )

    def al_desc(j, b):
        base = wid * _EPW + j * _CB
        return pltpu.make_async_copy(
            abufs[b], al_out.at[pl.ds(base, _CB)], sts[b]
        )

    for d in g_descs(0, 0):
        d.start()

    def do_chunk(j, b):
        @pl.when(j + 1 < _NCH)
        def _():
            for d in g_descs(j + 1, 1 - b):
                d.start()

        @pl.when(j >= 2)
        def _():
            al_desc(j - 2, b).wait()

        for d in g_descs(j, b):
            d.wait()
        gr, ech, abuf = grs[b], echs[b], abufs[b]

        @plsc.parallel_loop(0, _CB, unroll=8)
        def edge(t):
            al = ech[t, :] * gr[t, pl.ds(64, 16)]
            abuf[t, :] = al
            for q in range(4):
                vbuf[t, pl.ds(16 * q, 16)] = gr[t, pl.ds(16 * q, 16)] * al

        pltpu.sync_copy(vbuf, acc.at[icol.at[j]], add=True)
        al_desc(j, b).start()

    _run_ring(do_chunk)
    al_desc(_NCH - 2, 1).wait()
    al_desc(_NCH - 1, 0).wait()
    plsc.subcore_barrier()
    _copy_out(acc, po_out, cid, sid)


_sc_b = pl.kernel(
    _body_b,
    out_type=(
        jax.ShapeDtypeStruct((_K, 16), _f32),
        jax.ShapeDtypeStruct((_NC, _N, 64), _f32),
    ),
    mesh=_mesh,
    compiler_params=pltpu.CompilerParams(use_tc_tiling_on_sc=False),
    scratch_types=[
        pltpu.VMEM((_NCH, _CB), jnp.int32),
        pltpu.VMEM((_NCH, _CB), jnp.int32),
        pltpu.VMEM((_CB, 80), _f32),
        pltpu.VMEM((_CB, 80), _f32),
        pltpu.VMEM((_CB, 16), _f32),
        pltpu.VMEM((_CB, 16), _f32),
        pltpu.VMEM((_CB, 16), _f32),
        pltpu.VMEM((_CB, 16), _f32),
        pltpu.VMEM((_CB, 64), _f32),
        pltpu.VMEM((_RPS, 64), _f32),
        pltpu.VMEM_SHARED((_N, 64), _f32),
        pltpu.SemaphoreType.DMA,
        pltpu.SemaphoreType.DMA,
        pltpu.SemaphoreType.DMA,
        pltpu.SemaphoreType.DMA,
    ],
)


def _body_c(rowm, colm, oetab, al_hbm, pc_out,
            irow, icol, go0, go1, ach0, ach1, vbuf, zbuf, acc, sg0, sg1):
    wid, cid, sid = _wid_cid_sid()
    gos = (go0, go1)
    achs = (ach0, ach1)
    sgs = (sg0, sg1)

    _zero_fill(zbuf, _RPS)
    _zero_acc(zbuf, acc, sid)
    plsc.subcore_barrier()

    pltpu.sync_copy(rowm.at[wid], irow)
    pltpu.sync_copy(colm.at[wid], icol)

    def g_descs(j, b):
        base = wid * _EPW + j * _CB
        return (
            pltpu.make_async_copy(oetab.at[icol.at[j]], gos[b], sgs[b]),
            pltpu.make_async_copy(
                al_hbm.at[pl.ds(base, _CB)], achs[b], sgs[b]
            ),
        )

    for d in g_descs(0, 0):
        d.start()

    def do_chunk(j, b):
        @pl.when(j + 1 < _NCH)
        def _():
            for d in g_descs(j + 1, 1 - b):
                d.start()

        for d in g_descs(j, b):
            d.wait()
        go, ach = gos[b], achs[b]

        @plsc.parallel_loop(0, _CB, unroll=8)
        def edge(t):
            al = ach[t, :]
            for q in range(4):
                vbuf[t, pl.ds(16 * q, 16)] = go[t, pl.ds(16 * q, 16)] * al

        pltpu.sync_copy(vbuf, acc.at[irow.at[j]], add=True)

    _run_ring(do_chunk)
    plsc.subcore_barrier()
    _copy_out(acc, pc_out, cid, sid)


_sc_c = pl.kernel(
    _body_c,
    out_type=jax.ShapeDtypeStruct((_NC, _N, 64), _f32),
    mesh=_mesh,
    compiler_params=pltpu.CompilerParams(use_tc_tiling_on_sc=False),
    scratch_types=[
        pltpu.VMEM((_NCH, _CB), jnp.int32),
        pltpu.VMEM((_NCH, _CB), jnp.int32),
        pltpu.VMEM((_CB, 64), _f32),
        pltpu.VMEM((_CB, 64), _f32),
        pltpu.VMEM((_CB, 16), _f32),
        pltpu.VMEM((_CB, 16), _f32),
        pltpu.VMEM((_CB, 64), _f32),
        pltpu.VMEM((_RPS, 64), _f32),
        pltpu.VMEM_SHARED((_N, 64), _f32),
        pltpu.SemaphoreType.DMA,
        pltpu.SemaphoreType.DMA,
    ],
)


def _body_g(gm, sm, tab, p_out, gib, sib, gbuf0, gbuf1, zbuf, acc, sg0, sg1):
    wid, cid, sid = _wid_cid_sid()
    gbufs = (gbuf0, gbuf1)
    sgs = (sg0, sg1)
    _zero_fill(zbuf, _RPS)
    _zero_acc(zbuf, acc, sid)
    plsc.subcore_barrier()

    pltpu.sync_copy(gm.at[wid], gib)
    pltpu.sync_copy(sm.at[wid], sib)

    def g_desc(j, b):
        return pltpu.make_async_copy(tab.at[gib.at[j]], gbufs[b], sgs[b])

    g_desc(0, 0).start()

    def do_chunk(j, b):
        @pl.when(j + 1 < _NCH)
        def _():
            g_desc(j + 1, 1 - b).start()

        g_desc(j, b).wait()
        pltpu.sync_copy(gbufs[b], acc.at[sib.at[j]], add=True)

    _run_ring(do_chunk)
    plsc.subcore_barrier()
    _copy_out(acc, p_out, cid, sid)


_sc_g = pl.kernel(
    _body_g,
    out_type=jax.ShapeDtypeStruct((_NC, _N, 16), _f32),
    mesh=_mesh,
    compiler_params=pltpu.CompilerParams(use_tc_tiling_on_sc=False),
    scratch_types=[
        pltpu.VMEM((_NCH, _CB), jnp.int32),
        pltpu.VMEM((_NCH, _CB), jnp.int32),
        pltpu.VMEM((_CB, 16), _f32),
        pltpu.VMEM((_CB, 16), _f32),
        pltpu.VMEM((_RPS, 16), _f32),
        pltpu.VMEM_SHARED((_N, 16), _f32),
        pltpu.SemaphoreType.DMA,
        pltpu.SemaphoreType.DMA,
    ],
)


def _tc1(x_ref, w1_ref, ai_ref, aj_ref, hw_ref, xw_ref, trow_ref, tcol_ref,
         htab_ref):
    xw = jnp.dot(x_ref[...], w1_ref[...], preferred_element_type=_f32)
    xw_ref[...] = xw
    si = jnp.dot(xw, ai_ref[...], preferred_element_type=_f32)
    sj = jnp.dot(xw, aj_ref[...], preferred_element_type=_f32)
    trow_ref[...] = jnp.concatenate([si, si], axis=1)
    tcol_ref[...] = jnp.concatenate([sj, sj], axis=1)
    htab_ref[...] = jnp.concatenate(
        [jnp.zeros((_N, 8), _f32), hw_ref[...], jnp.zeros((_N, 7), _f32)],
        axis=1,
    )


def _tc2(pa_ref, pb_ref, xw_ref, r_ref, db_ref):
    sa = pa_ref[0] + pa_ref[1]
    ssum = sa[:, :8]
    d = sa[:, 8:9]
    b = pb_ref[0][:, 0:1] + pb_ref[1][:, 0:1]
    inv = 1.0 / (ssum + 1e-16)
    dinv = jnp.where(d > 0, 1.0 / jnp.where(d > 0, d, 1.0), 0.0)
    binv = jnp.where(b > 0, 1.0 / jnp.where(b > 0, b, 1.0), 0.0)
    r_ref[...] = jnp.concatenate([xw_ref[...], inv, inv], axis=1)
    db_ref[...] = jnp.concatenate(
        [dinv, binv, jnp.zeros((_N, 6), _f32)], axis=1
    )


def _tc3(po_ref, db_ref, oe_ref):
    oe_ref[...] = (po_ref[0] + po_ref[1]) * db_ref[...][:, 1:2]


def _tc4(pc_ref, db_ref, b1_ref, w2_ref, h2_ref):
    s = (pc_ref[0] + pc_ref[1]) * db_ref[...][:, 0:1] + b1_ref[...]
    h = jnp.where(s > 0, s, jnp.exp(jnp.minimum(s, 0.0)) - 1.0)
    hh = jnp.dot(h, w2_ref[...], preferred_element_type=_f32)
    h2_ref[...] = jnp.concatenate(
        [hh, jnp.zeros((_N, 16 - _NCLS), _f32)], axis=1
    )


def _tc5(pd_ref, db_ref, oe2_ref):
    oe2_ref[...] = (pd_ref[0] + pd_ref[1]) * db_ref[...][:, 1:2]


def _tc6(pe_ref, db_ref, b2_ref, out_ref):
    s = (pe_ref[0] + pe_ref[1])[:, :_NCLS] * db_ref[...][:, 0:1] + b2_ref[...]
    m = jnp.max(s, axis=1, keepdims=True)
    zz = s - m
    out_ref[...] = zz - jnp.log(jnp.sum(jnp.exp(zz), axis=1, keepdims=True))


def _call_tc(fn, out_shape, *args):
    return pl.pallas_call(fn, out_shape=out_shape)(*args)


def kernel(x, edge_index, hyperedge_weight, W1, att1, b1, W2, b2):
    rowm = edge_index[0].reshape(_NW, _NCH, _CB)
    colm = edge_index[1].reshape(_NW, _NCH, _CB)
    hw = hyperedge_weight.reshape(_N, 1)
    perm = jnp.arange(64).reshape(_HEADS, _HID).T.reshape(64)
    w1p = W1[:, perm]
    b1p = b1[perm]
    w2p = W2[perm, :]
    eye = jnp.eye(_HEADS, dtype=_f32)
    a_i = (att1[0, :, :_HID].T[:, :, None] * eye[None, :, :]).reshape(64, 8)
    a_j = (att1[0, :, _HID:].T[:, :, None] * eye[None, :, :]).reshape(64, 8)

    xw, trow, tcol, htab = _call_tc(
        _tc1,
        (
            jax.ShapeDtypeStruct((_N, 64), _f32),
            jax.ShapeDtypeStruct((_N, 16), _f32),
            jax.ShapeDtypeStruct((_N, 16), _f32),
            jax.ShapeDtypeStruct((_N, 16), _f32),
        ),
        x, w1p, a_i, a_j, hw,
    )

    e, pa, pb = _sc_a(rowm, colm, trow, tcol, htab)

    rtab, db = _call_tc(
        _tc2,
        (
            jax.ShapeDtypeStruct((_N, 80), _f32),
            jax.ShapeDtypeStruct((_N, 8), _f32),
        ),
        pa, pb, xw,
    )

    alpha, po = _sc_b(rowm, colm, rtab, e)

    oe = _call_tc(_tc3, jax.ShapeDtypeStruct((_N, 64), _f32), po, db)

    pc = _sc_c(rowm, colm, oe, alpha)

    h2 = _call_tc(
        _tc4, jax.ShapeDtypeStruct((_N, 16), _f32),
        pc, db, b1p.reshape(1, -1), w2p,
    )

    pd = _sc_g(rowm, colm, h2)

    oe2 = _call_tc(_tc5, jax.ShapeDtypeStruct((_N, 16), _f32), pd, db)

    pe = _sc_g(colm, rowm, oe2)

    out = _call_tc(
        _tc6, jax.ShapeDtypeStruct((_N, _NCLS), _f32),
        pe, db, b2.reshape(1, -1),
    )
    return out
